# baseline (speedup 1.0000x reference)
import functools

import jax
import jax.numpy as jnp
from jax import lax
from jax.experimental import pallas as pl
from jax.experimental.pallas import tpu as pltpu

_CTX = 4
_EMBED = 64
_HID = 128
_BLK = 16384


def _pass1_body(v_total, blk, idx_ref, table_ref, w1_ref, b1_ref, w2_ref,
                b2_ref, out_ref, bm_ref, bs_ref, emb_ref, h_ref, sem):
    i = pl.program_id(0)

    @pl.when(i == 0)
    def _():
        copies = [
            pltpu.make_async_copy(
                table_ref.at[pl.ds(idx_ref[j], 1), :],
                emb_ref.at[pl.ds(j, 1), :],
                sem,
            )
            for j in range(_CTX)
        ]
        for c in copies:
            c.start()
        for c in copies:
            c.wait()
        acc = b1_ref[...]
        for j in range(_CTX):
            acc = acc + jnp.dot(emb_ref[pl.ds(j, 1), :], w1_ref[j],
                                preferred_element_type=jnp.float32)
        h_ref[0:1, :] = jnp.maximum(acc, 0.0)

    logits = jnp.dot(h_ref[0:1, :], w2_ref[...],
                     preferred_element_type=jnp.float32) + b2_ref[...]
    col = i * blk + lax.broadcasted_iota(jnp.int32, (1, blk), 1)
    masked = jnp.where(col < v_total, logits, -jnp.inf)
    bm = jnp.max(masked)
    bs = jnp.sum(jnp.exp(masked - bm))
    out_ref[...] = logits
    bm_ref[...] = jnp.full((1, 1, 128), bm, jnp.float32)
    bs_ref[...] = jnp.full((1, 1, 128), bs, jnp.float32)


def _pass2_body(bm_ref, bs_ref, logits_ref, out_ref):
    bm = bm_ref[...]
    bs = bs_ref[...]
    m = jnp.max(bm)
    s = jnp.sum(bs * jnp.exp(bm - m)) * (1.0 / 128.0)
    lse = m + jnp.log(s)
    out_ref[...] = logits_ref[...] - lse


def kernel(inputs, table, W1, b1, W2, b2):
    v_total = W2.shape[1]
    nb = pl.cdiv(v_total, _BLK)

    w1r = W1.reshape(_CTX, _EMBED, _HID)
    b1r = b1.reshape(1, _HID)
    b2r = b2.reshape(1, v_total)

    logits, bm, bs = pl.pallas_call(
        functools.partial(_pass1_body, v_total, _BLK),
        grid_spec=pltpu.PrefetchScalarGridSpec(
            num_scalar_prefetch=1,
            grid=(nb,),
            in_specs=[
                pl.BlockSpec(memory_space=pltpu.MemorySpace.HBM),
                pl.BlockSpec((_CTX, _EMBED, _HID), lambda i, idx: (0, 0, 0)),
                pl.BlockSpec((1, _HID), lambda i, idx: (0, 0)),
                pl.BlockSpec((_HID, _BLK), lambda i, idx: (0, i)),
                pl.BlockSpec((1, _BLK), lambda i, idx: (0, i)),
            ],
            out_specs=[
                pl.BlockSpec((1, _BLK), lambda i, idx: (0, i)),
                pl.BlockSpec((1, 1, 128), lambda i, idx: (i, 0, 0)),
                pl.BlockSpec((1, 1, 128), lambda i, idx: (i, 0, 0)),
            ],
            scratch_shapes=[
                pltpu.VMEM((_CTX, _EMBED), jnp.float32),
                pltpu.VMEM((8, _HID), jnp.float32),
                pltpu.SemaphoreType.DMA,
            ],
        ),
        out_shape=[
            jax.ShapeDtypeStruct((1, v_total), jnp.float32),
            jax.ShapeDtypeStruct((nb, 1, 128), jnp.float32),
            jax.ShapeDtypeStruct((nb, 1, 128), jnp.float32),
        ],
        compiler_params=pltpu.CompilerParams(
            dimension_semantics=("arbitrary",),
        ),
    )(inputs, table, w1r, b1r, W2, b2r)

    out = pl.pallas_call(
        _pass2_body,
        in_specs=[
            pl.BlockSpec((nb, 1, 128), lambda: (0, 0, 0)),
            pl.BlockSpec((nb, 1, 128), lambda: (0, 0, 0)),
            pl.BlockSpec((1, v_total), lambda: (0, 0)),
        ],
        out_specs=pl.BlockSpec((1, v_total), lambda: (0, 0)),
        out_shape=jax.ShapeDtypeStruct((1, v_total), jnp.float32),
    )(bm, bs, logits)

    return out


# speedup vs baseline: 1.0090x; 1.0090x over previous
import functools

import jax
import jax.numpy as jnp
from jax import lax
from jax.experimental import pallas as pl
from jax.experimental.pallas import tpu as pltpu

_CTX = 4
_EMBED = 64
_HID = 128
_BLK = 32768


def _pass1_body(v_total, blk, idx_ref, table_ref, w1_ref, b1_ref, w2_ref,
                b2_ref, out_ref, bm_ref, bs_ref, emb_ref, h_ref, sem):
    i = pl.program_id(0)

    @pl.when(i == 0)
    def _():
        copies = [
            pltpu.make_async_copy(
                table_ref.at[pl.ds(idx_ref[j], 1), :],
                emb_ref.at[pl.ds(j, 1), :],
                sem,
            )
            for j in range(_CTX)
        ]
        for c in copies:
            c.start()
        for c in copies:
            c.wait()
        acc = b1_ref[...]
        for j in range(_CTX):
            acc = acc + jnp.dot(emb_ref[pl.ds(j, 1), :], w1_ref[j],
                                preferred_element_type=jnp.float32)
        h_ref[0:1, :] = jnp.maximum(acc, 0.0)

    logits = jnp.dot(h_ref[0:1, :], w2_ref[...],
                     preferred_element_type=jnp.float32) + b2_ref[...]
    col = i * blk + lax.broadcasted_iota(jnp.int32, (1, blk), 1)
    masked = jnp.where(col < v_total, logits, -jnp.inf)
    bm = jnp.max(masked)
    bs = jnp.sum(jnp.exp(masked - bm))
    out_ref[...] = logits
    bm_ref[...] = jnp.full((1, 1, 128), bm, jnp.float32)
    bs_ref[...] = jnp.full((1, 1, 128), bs, jnp.float32)


def _pass2_body(bm_ref, bs_ref, logits_ref, out_ref):
    bm = bm_ref[...]
    bs = bs_ref[...]
    m = jnp.max(bm)
    s = jnp.sum(bs * jnp.exp(bm - m)) * (1.0 / 128.0)
    lse = m + jnp.log(s)
    out_ref[...] = logits_ref[...] - lse


def kernel(inputs, table, W1, b1, W2, b2):
    v_total = W2.shape[1]
    nb = pl.cdiv(v_total, _BLK)

    w1r = W1.reshape(_CTX, _EMBED, _HID)
    b1r = b1.reshape(1, _HID)
    b2r = b2.reshape(1, v_total)

    logits, bm, bs = pl.pallas_call(
        functools.partial(_pass1_body, v_total, _BLK),
        grid_spec=pltpu.PrefetchScalarGridSpec(
            num_scalar_prefetch=1,
            grid=(nb,),
            in_specs=[
                pl.BlockSpec(memory_space=pltpu.MemorySpace.HBM),
                pl.BlockSpec((_CTX, _EMBED, _HID), lambda i, idx: (0, 0, 0)),
                pl.BlockSpec((1, _HID), lambda i, idx: (0, 0)),
                pl.BlockSpec((_HID, _BLK), lambda i, idx: (0, i)),
                pl.BlockSpec((1, _BLK), lambda i, idx: (0, i)),
            ],
            out_specs=[
                pl.BlockSpec((1, _BLK), lambda i, idx: (0, i)),
                pl.BlockSpec((1, 1, 128), lambda i, idx: (i, 0, 0)),
                pl.BlockSpec((1, 1, 128), lambda i, idx: (i, 0, 0)),
            ],
            scratch_shapes=[
                pltpu.VMEM((_CTX, _EMBED), jnp.float32),
                pltpu.VMEM((8, _HID), jnp.float32),
                pltpu.SemaphoreType.DMA,
            ],
        ),
        out_shape=[
            jax.ShapeDtypeStruct((1, v_total), jnp.float32),
            jax.ShapeDtypeStruct((nb, 1, 128), jnp.float32),
            jax.ShapeDtypeStruct((nb, 1, 128), jnp.float32),
        ],
        compiler_params=pltpu.CompilerParams(
            dimension_semantics=("arbitrary",),
        ),
    )(inputs, table, w1r, b1r, W2, b2r)

    out = pl.pallas_call(
        _pass2_body,
        in_specs=[
            pl.BlockSpec((nb, 1, 128), lambda: (0, 0, 0)),
            pl.BlockSpec((nb, 1, 128), lambda: (0, 0, 0)),
            pl.BlockSpec((1, v_total), lambda: (0, 0)),
        ],
        out_specs=pl.BlockSpec((1, v_total), lambda: (0, 0)),
        out_shape=jax.ShapeDtypeStruct((1, v_total), jnp.float32),
    )(bm, bs, logits)

    return out
